# Initial kernel scaffold; baseline (speedup 1.0000x reference)
import jax
import jax.numpy as jnp
from jax.experimental import pallas as pl


def kernel(feature, frame_level_feature, segments, frame_segments, conf_result_feature, w_cur, b_cur, g_cur, be_cur, w_lr, b_lr, g_lr, be_lr, w_roi, b_roi, g_roi, be_roi, w_prop, b_prop, g_prop, be_prop):
    raise NotImplementedError("write your pallas kernel here")


# speedup vs baseline: 1.5800x; 1.5800x over previous
import functools

import jax
import jax.numpy as jnp
from jax.experimental import pallas as pl
from jax.experimental.pallas import tpu as pltpu

B, C, T, TF, CCONF = 8, 512, 64, 256, 400
GROUPS, EPS = 32, 1e-5
NCORES = 2
BPC = B // NCORES
RT = BPC * T
RF = BPC * TF
CPAD = 512
NEG = -3e38


def _gn_relu(y, r_mat, exp_mat, p_mat, pt_mat, gamma, beta, cpg):
    dot = functools.partial(jnp.dot, preferred_element_type=jnp.float32)
    s_b = dot(r_mat, y)
    q_b = dot(r_mat, y * y)
    cnt = float(cpg * T)
    mu = dot(s_b, p_mat) * (1.0 / cnt)
    q = dot(q_b, p_mat) * (1.0 / cnt)
    rs = jax.lax.rsqrt(jnp.maximum(q - mu * mu, 0.0) + EPS)
    a4 = dot(rs, pt_mat) * gamma
    c4 = beta - dot(mu * rs, pt_mat) * gamma
    a_full = dot(exp_mat, a4)
    c_full = dot(exp_mat, c4)
    return jnp.maximum(y * a_full + c_full, 0.0)


def _pool(src, seg_ref, seg_base, tin, cn, out_scr):
    half = cn // 2
    tiota = jax.lax.broadcasted_iota(jnp.int32, (tin, half), 0)
    for b in range(BPC):
        xb = src[b * tin:(b + 1) * tin, :]
        xl = xb[:, :half]
        xr = xb[:, half:]

        def group_body(gi, _, b=b, xl=xl, xr=xr):
            rows = []
            for j in range(8):
                n = (b * T + gi * 8 + j) * 4 + seg_base
                lo1 = seg_ref[n]
                hi1 = seg_ref[n + 1]
                lo2 = seg_ref[n + 2]
                hi2 = seg_ref[n + 3]
                ml = (tiota >= lo1) & (tiota <= hi1)
                mr = (tiota >= lo2) & (tiota <= hi2)
                vl = jnp.max(jnp.where(ml, xl, NEG), axis=0, keepdims=True)
                vr = jnp.max(jnp.where(mr, xr, NEG), axis=0, keepdims=True)
                rows.append(jnp.concatenate([vl, vr], axis=1))
            tile = jnp.concatenate(rows, axis=0)
            out_scr[pl.ds(b * T + gi * 8, 8), :] = tile
            return 0

        jax.lax.fori_loop(0, T // 8, group_body, 0)


def _body(seg_ref, x_ref, f_ref, conf_ref,
          wcur_ref, wlr_ref, wroi_ref, wp1_ref, wp2_ref, wp3_ref, wp4_ref,
          bcur_ref, gcur_ref, becur_ref, blr_ref, glr_ref, belr_ref,
          broi_ref, groi_ref, beroi_ref, bprop_ref, gprop_ref, beprop_ref,
          r_ref, exp_ref, p512_ref, pt512_ref, p1024_ref, pt1024_ref,
          out_ref, f2_ref, pf_scr, pr_scr):
    core = pl.program_id(0)
    dot = functools.partial(jnp.dot, preferred_element_type=jnp.float32)
    x = x_ref[0]
    r_mat = r_ref[...]
    exp_mat = exp_ref[...]
    p512 = p512_ref[...]
    pt512 = pt512_ref[...]
    p1024 = p1024_ref[...]
    pt1024 = pt1024_ref[...]

    cur = _gn_relu(dot(x, wcur_ref[...]) + bcur_ref[...], r_mat, exp_mat,
                   p512, pt512, gcur_ref[...], becur_ref[...], C // GROUPS)
    f2 = _gn_relu(dot(x, wlr_ref[...]) + blr_ref[...], r_mat, exp_mat,
                  p1024, pt1024, glr_ref[...], belr_ref[...], 2 * C // GROUPS)
    f2_ref[0] = f2

    seg_off = core * (RT * 8)
    _pool(f2, seg_ref, seg_off, T, 2 * C, pf_scr)
    _pool(f_ref[0], seg_ref, seg_off + RT * 4, TF, C, pr_scr)

    pf = pf_scr[...]
    roi = _gn_relu(dot(pr_scr[...], wroi_ref[...]) + broi_ref[...], r_mat,
                   exp_mat, p512, pt512, groi_ref[...], beroi_ref[...],
                   C // GROUPS)
    y = (dot(roi, wp1_ref[...]) + dot(pf, wp2_ref[...])
         + dot(cur, wp3_ref[...]) + dot(conf_ref[0], wp4_ref[...])
         + bprop_ref[...])
    out_ref[0] = _gn_relu(y, r_mat, exp_mat, p512, pt512, gprop_ref[...],
                          beprop_ref[...], C // GROUPS)


def kernel(feature, frame_level_feature, segments, frame_segments,
           conf_result_feature, w_cur, b_cur, g_cur, be_cur, w_lr, b_lr,
           g_lr, be_lr, w_roi, b_roi, g_roi, be_roi, w_prop, b_prop,
           g_prop, be_prop):
    f32 = jnp.float32
    x = feature.transpose(0, 2, 1).reshape(NCORES, RT, C)
    f = frame_level_feature.transpose(0, 2, 1).reshape(NCORES, RF, C)
    conf = conf_result_feature.transpose(0, 2, 1)
    conf = jnp.pad(conf, ((0, 0), (0, 0), (0, CPAD - CCONF)))
    conf = conf.reshape(NCORES, RT, CPAD)

    s = jnp.clip(jnp.floor(segments), 0, T - 1).astype(jnp.int32)
    fs = jnp.clip(jnp.floor(frame_segments), 0, TF - 1).astype(jnp.int32)
    seg_flat = jnp.concatenate(
        [s.reshape(NCORES, RT * 4), fs.reshape(NCORES, RT * 4)],
        axis=1).reshape(-1)

    wcur = w_cur.T
    wlr = w_lr.T
    wroi = w_roi.T
    wpt = w_prop.T
    wp1 = wpt[0:C]
    wp2 = wpt[C:3 * C]
    wp3 = wpt[3 * C:4 * C]
    wp4 = jnp.pad(wpt[4 * C:], ((0, CPAD - CCONF), (0, 0)))

    cg = jnp.arange(C) // (C // GROUPS)
    cg2 = jnp.arange(2 * C) // (2 * C // GROUPS)
    p512 = (cg[:, None] == jnp.arange(GROUPS)[None, :]).astype(f32)
    p1024 = (cg2[:, None] == jnp.arange(GROUPS)[None, :]).astype(f32)
    rb = jnp.arange(RT) // T
    r_mat = (jnp.arange(BPC)[:, None] == rb[None, :]).astype(f32)
    exp_mat = r_mat.T

    row = lambda v: v[None, :].astype(f32)

    grid_spec = pltpu.PrefetchScalarGridSpec(
        num_scalar_prefetch=1,
        grid=(NCORES,),
        in_specs=[
            pl.BlockSpec((1, RT, C), lambda i, s: (i, 0, 0)),
            pl.BlockSpec((1, RF, C), lambda i, s: (i, 0, 0)),
            pl.BlockSpec((1, RT, CPAD), lambda i, s: (i, 0, 0)),
        ] + [pl.BlockSpec(w.shape, lambda i, s, nd=w.ndim: (0,) * nd) for w in
             (wcur, wlr, wroi, wp1, wp2, wp3, wp4,
              row(b_cur), row(g_cur), row(be_cur),
              row(b_lr), row(g_lr), row(be_lr),
              row(b_roi), row(g_roi), row(be_roi),
              row(b_prop), row(g_prop), row(be_prop),
              r_mat, exp_mat, p512, p512.T, p1024, p1024.T)],
        out_specs=[
            pl.BlockSpec((1, RT, C), lambda i, s: (i, 0, 0)),
            pl.BlockSpec((1, RT, 2 * C), lambda i, s: (i, 0, 0)),
        ],
        scratch_shapes=[
            pltpu.VMEM((RT, 2 * C), f32),
            pltpu.VMEM((RT, C), f32),
        ],
    )

    out, f2 = pl.pallas_call(
        _body,
        grid_spec=grid_spec,
        out_shape=[
            jax.ShapeDtypeStruct((NCORES, RT, C), f32),
            jax.ShapeDtypeStruct((NCORES, RT, 2 * C), f32),
        ],
        compiler_params=pltpu.CompilerParams(
            dimension_semantics=("parallel",),
            vmem_limit_bytes=100 * 1024 * 1024,
        ),
    )(seg_flat, x, f, conf, wcur, wlr, wroi, wp1, wp2, wp3, wp4,
      row(b_cur), row(g_cur), row(be_cur),
      row(b_lr), row(g_lr), row(be_lr),
      row(b_roi), row(g_roi), row(be_roi),
      row(b_prop), row(g_prop), row(be_prop),
      r_mat, exp_mat, p512, p512.T, p1024, p1024.T)

    out = out.reshape(B, T, C).transpose(0, 2, 1)
    f2 = f2.reshape(B, T, 2 * C).transpose(0, 2, 1)
    return (out, f2)
